# baseline (speedup 1.0000x reference)
import functools

import jax
import jax.numpy as jnp
from jax import lax
from jax.experimental import pallas as pl
from jax.experimental.pallas import tpu as pltpu
from jax.experimental.pallas import tpu_sc as plsc

VOCAB = 1000000
EMBED_DIM = 32
BATCH = 16384
HIST = 50

NW = 32
BB = 256
N_BC = BATCH // BB
N_CID = HIST * N_BC
PER_W = N_CID // NW
SETS = 4

_mesh = plsc.VectorSubcoreMesh(core_axis_name="c", subcore_axis_name="s")

_i32 = jnp.int32


@functools.partial(
    pl.kernel,
    out_type=jax.ShapeDtypeStruct((HIST, EMBED_DIM, BATCH), jnp.float32),
    mesh=_mesh,
    scratch_types=(
        [pltpu.VMEM((BB,), _i32) for _ in range(SETS)]
        + [pltpu.VMEM((BB, EMBED_DIM), jnp.float32) for _ in range(SETS)]
        + [pltpu.VMEM((EMBED_DIM, BB), jnp.float32) for _ in range(SETS)]
        + [pltpu.SemaphoreType.DMA for _ in range(SETS)]
        + [pltpu.SemaphoreType.DMA for _ in range(SETS)]
    ),
    compiler_params=pltpu.CompilerParams(use_tc_tiling_on_sc=False,
                                         needs_layout_passes=False),
)
def _sc_embed(pt_t, tab_hbm, out_p, *bufs):
    c = lax.axis_index("c")
    s = lax.axis_index("s")
    w = s * 2 + c

    idx = bufs[0:SETS]
    rows = bufs[SETS:2 * SETS]
    oblk = bufs[2 * SETS:3 * SETS]
    gsem = bufs[3 * SETS:4 * SETS]
    osem = bufs[4 * SETS:5 * SETS]

    def cid_of(j):
        return w * PER_W + j

    def idx_load(j, p):
        cid = cid_of(j)
        h = cid // N_BC
        b0 = pl.multiple_of((cid % N_BC) * BB, 128)
        pltpu.sync_copy(pt_t.at[h, pl.ds(b0, BB)], idx[p])

    def gather(p):
        return pltpu.make_async_copy(tab_hbm.at[idx[p]], rows[p], gsem[p])

    def outw(j, p):
        cid = cid_of(j)
        h = cid // N_BC
        b0 = pl.multiple_of((cid % N_BC) * BB, 128)
        return pltpu.make_async_copy(oblk[p],
                                     out_p.at[h, :, pl.ds(b0, BB)],
                                     osem[p])

    def transform(p):
        def body(bt, _):
            bvec = lax.iota(_i32, 16) + 16 * bt
            for d in range(EMBED_DIM):
                dvec = jnp.full((16,), d, _i32)
                vec = plsc.load_gather(rows[p], [bvec, dvec])
                oblk[p][d, pl.ds(16 * bt, 16)] = vec
            return 0

        lax.fori_loop(0, BB // 16, body, 0, unroll=False)

    n_grp = PER_W // SETS

    for q in range(SETS):
        idx_load(q, q)
        gather(q).start()

    def group(t, _):
        for q in range(SETS):
            j = SETS * t + q
            gather(q).wait()

            @pl.when(t > 0)
            def _():
                outw(j - SETS, q).wait()

            transform(q)
            outw(j, q).start()

            @pl.when(t + 1 < n_grp)
            def _():
                idx_load(j + SETS, q)
                gather(q).start()

        return 0

    lax.fori_loop(0, n_grp, group, 0, unroll=False)
    for q in range(SETS):
        outw(PER_W - SETS + q, q).wait()


def kernel(PlayType, table):
    pt_t = PlayType.T
    out_p = _sc_embed(pt_t, table)
    return out_p.transpose(2, 0, 1)


# speedup vs baseline: 1.0413x; 1.0413x over previous
import functools

import jax
import jax.numpy as jnp
from jax import lax
from jax.experimental import pallas as pl
from jax.experimental.pallas import tpu as pltpu
from jax.experimental.pallas import tpu_sc as plsc

VOCAB = 1000000
EMBED_DIM = 32
BATCH = 16384
HIST = 50

NW = 32
BB = 512
N_BC = BATCH // BB
N_CID = HIST * N_BC
PER_W = N_CID // NW

_mesh = plsc.VectorSubcoreMesh(core_axis_name="c", subcore_axis_name="s")

_i32 = jnp.int32


@functools.partial(
    pl.kernel,
    out_type=jax.ShapeDtypeStruct((HIST, EMBED_DIM, BATCH), jnp.float32),
    mesh=_mesh,
    scratch_types=[
        pltpu.VMEM((PER_W * BB,), _i32),
        pltpu.VMEM((BB, EMBED_DIM), jnp.float32),
        pltpu.VMEM((BB, EMBED_DIM), jnp.float32),
        pltpu.VMEM((EMBED_DIM, BB), jnp.float32),
        pltpu.VMEM((EMBED_DIM, BB), jnp.float32),
        pltpu.SemaphoreType.DMA,
        pltpu.SemaphoreType.DMA,
        pltpu.SemaphoreType.DMA,
        pltpu.SemaphoreType.DMA,
    ],
    compiler_params=pltpu.CompilerParams(use_tc_tiling_on_sc=False,
                                         needs_layout_passes=False),
)
def _sc_embed(pt_flat, tab_hbm, out_p, idxall, rows_a, rows_b,
              oblk_a, oblk_b, gsa, gsb, osa, osb):
    c = lax.axis_index("c")
    s = lax.axis_index("s")
    w = s * 2 + c

    rows = (rows_a, rows_b)
    oblk = (oblk_a, oblk_b)
    gsem = (gsa, gsb)
    osem = (osa, osb)

    def cid_of(j):
        return w * PER_W + j

    pltpu.sync_copy(pt_flat.at[pl.ds(pl.multiple_of(w * PER_W * BB, 128),
                                     PER_W * BB)], idxall)

    def gather(j, p):
        return pltpu.make_async_copy(
            tab_hbm.at[idxall.at[pl.ds(j * BB, BB)]], rows[p], gsem[p])

    def outw(j, p):
        cid = cid_of(j)
        h = cid // N_BC
        b0 = pl.multiple_of((cid % N_BC) * BB, 128)
        return pltpu.make_async_copy(oblk[p],
                                     out_p.at[h, :, pl.ds(b0, BB)],
                                     osem[p])

    def transform(p):
        def body(bt, _):
            bvec = lax.iota(_i32, 16) + 16 * bt
            for d in range(EMBED_DIM):
                dvec = jnp.full((16,), d, _i32)
                vec = plsc.load_gather(rows[p], [bvec, dvec])
                oblk[p][d, pl.ds(16 * bt, 16)] = vec
            return 0

        lax.fori_loop(0, BB // 16, body, 0, unroll=False)

    n_pair = PER_W // 2

    gather(0, 0).start()
    gather(1, 1).start()

    def pair(t, _):
        j0 = 2 * t
        j1 = j0 + 1

        gather(j0, 0).wait()

        @pl.when(t > 0)
        def _():
            outw(j0 - 2, 0).wait()

        transform(0)
        outw(j0, 0).start()

        @pl.when(t + 1 < n_pair)
        def _():
            gather(j0 + 2, 0).start()

        gather(j1, 1).wait()

        @pl.when(t > 0)
        def _():
            outw(j1 - 2, 1).wait()

        transform(1)
        outw(j1, 1).start()

        @pl.when(t + 1 < n_pair)
        def _():
            gather(j1 + 2, 1).start()

        return 0

    lax.fori_loop(0, n_pair, pair, 0, unroll=False)
    outw(PER_W - 2, 0).wait()
    outw(PER_W - 1, 1).wait()


def kernel(PlayType, table):
    pt_flat = PlayType.T.reshape(HIST * BATCH)
    out_p = _sc_embed(pt_flat, table)
    return out_p.transpose(2, 0, 1)
